# baseline (speedup 1.0000x reference)
import functools

import jax
import jax.numpy as jnp
from jax import lax
from jax.experimental import pallas as pl
from jax.experimental.pallas import tpu as pltpu
from jax.experimental.pallas import tpu_sc as plsc

N = 10000
E = 320000
DIN = 128
HID = 16
DOUT = 16
K = 3

NT = 16
NPAD = 10240
NODES_T = NPAD // NT
EDGES_T = E // NT
W_E = 1000
NWIN = EDGES_T // W_E
NBUF = 2


def _fill_rows(ref, n_rows, value):
    def body(i, _):
        ref[i, :] = jnp.full((16,), value, jnp.float32)
        return 0
    lax.fori_loop(0, n_rows, body, 0)


def _make_sc_layer(compute_deg: bool, fdim: int):
    mesh = plsc.VectorSubcoreMesh(core_axis_name="c", subcore_axis_name="s",
                                  num_cores=2, num_subcores=NT)
    out_type = [jax.ShapeDtypeStruct((NPAD, fdim), jnp.float32),
                jax.ShapeDtypeStruct((NPAD, 16), jnp.float32)]
    if compute_deg:
        out_type.append(jax.ShapeDtypeStruct((NPAD, 16), jnp.float32))

    scratch = [
        pltpu.VMEM((NWIN, W_E), jnp.int32),
        pltpu.VMEM((NWIN, W_E), jnp.int32),
        pltpu.VMEM((NBUF, W_E, 16), jnp.float32),
        pltpu.VMEM((NODES_T, 16), jnp.float32),
        pltpu.VMEM((NODES_T, 16), jnp.float32),
        pltpu.VMEM((NODES_T, 16), jnp.float32),
        pltpu.VMEM((NODES_T, 16), jnp.float32),
        pltpu.VMEM((16,), jnp.float32),
        pltpu.VMEM_SHARED((NPAD, 16), jnp.float32),
    ] + [pltpu.SemaphoreType.DMA] * (2 * NBUF + 1)

    def body(*refs):
        if compute_deg:
            (z0, z1, z2, z3, row_h, col_h, b_h,
             out_h, u_hbm, drep_h,
             row_v, col_v, gbufs, accb, zbuf, ubuf, drepb, bbuf,
             acc_sp, *sems) = refs
        else:
            (z0, z1, z2, z3, row_h, col_h, b_h, drep_in,
             out_h, u_hbm,
             row_v, col_v, gbufs, accb, zbuf, ubuf, drepb, bbuf,
             acc_sp, *sems) = refs
        zs = [z0, z1, z2, z3]
        gsems = sems[:NBUF]
        ssems = sems[NBUF:2 * NBUF]
        zsem = sems[2 * NBUF]
        cid = lax.axis_index("c")
        sid = lax.axis_index("s")
        base = sid * NODES_T
        nsl = pl.ds(base, NODES_T)

        rd = pltpu.async_copy(row_h.at[sid], row_v, gsems[0])
        cd = pltpu.async_copy(col_h.at[sid], col_v, gsems[1])
        bd = pltpu.async_copy(b_h, bbuf, zsem)
        if not compute_deg:
            dd = pltpu.async_copy(drep_in.at[nsl], drepb, ssems[0])

        _fill_rows(ubuf, NODES_T, 0.0)
        pltpu.sync_copy(ubuf, acc_sp.at[nsl])
        rd.wait()
        cd.wait()
        bd.wait()
        if not compute_deg:
            dd.wait()
        plsc.subcore_barrier()

        if compute_deg:
            ones = gbufs.at[0]
            _fill_rows(ones, W_E, 1.0)
            degd = [
                pltpu.async_copy(ones, acc_sp.at[col_v.at[j]], ssems[0],
                                 add=True)
                for j in range(NWIN)
            ]
            for d in degd:
                d.wait()
            plsc.subcore_barrier()
            pltpu.sync_copy(acc_sp.at[nsl], accb)

            def dinv_body(i, _):
                d = accb[i, :]
                bits = lax.bitcast_convert_type(d, jnp.int32)
                y = lax.bitcast_convert_type(
                    jnp.int32(0x5F3759DF) - (bits >> 1), jnp.float32)
                for _ in range(3):
                    y = y * (1.5 - 0.5 * d * y * y)
                drepb[i, :] = jnp.where(d > 0.5, y, 0.0)
                return 0
            lax.fori_loop(0, NODES_T, dinv_body, 0)

            @pl.when(cid == 0)
            def _():
                pltpu.sync_copy(drepb, drep_h.at[nsl])

            pltpu.sync_copy(ubuf, acc_sp.at[nsl])

        pltpu.sync_copy(zs[K].at[nsl], zbuf)

        def init_body(i, _):
            zbuf[i, :] = drepb[i, :] * zbuf[i, :]
            return 0
        lax.fori_loop(0, NODES_T, init_body, 0)
        pltpu.sync_copy(zbuf, u_hbm.at[nsl])
        plsc.subcore_barrier()

        for k in range(K - 1, -1, -1):
            zd = pltpu.async_copy(zs[k].at[nsl], zbuf, zsem)
            gd = [None] * NWIN
            sd = [None] * NWIN
            for j in range(min(NBUF, NWIN)):
                gd[j] = pltpu.async_copy(
                    u_hbm.at[row_v.at[j]], gbufs.at[j % NBUF], gsems[j % NBUF])
            for j in range(NWIN):
                b = j % NBUF
                gd[j].wait()
                sd[j] = pltpu.async_copy(
                    gbufs.at[b], acc_sp.at[col_v.at[j]], ssems[b], add=True)
                if j >= 1 and j + NBUF - 1 < NWIN:
                    sd[j - 1].wait()
                    nj = j + NBUF - 1
                    gd[nj] = pltpu.async_copy(
                        u_hbm.at[row_v.at[nj]], gbufs.at[nj % NBUF],
                        gsems[nj % NBUF])
            for j in range(max(0, NWIN - NBUF), NWIN):
                sd[j].wait()
            plsc.subcore_barrier()

            pltpu.sync_copy(acc_sp.at[nsl], accb)
            zd.wait()
            if k > 0:
                def hop_body(i, _):
                    d = drepb[i, :]
                    accb[i, :] = d * (zbuf[i, :] + d * accb[i, :])
                    return 0
                lax.fori_loop(0, NODES_T, hop_body, 0)
                ud = pltpu.async_copy(accb, u_hbm.at[nsl], gsems[0])
                zc = pltpu.async_copy(ubuf, acc_sp.at[nsl], gsems[1])
                ud.wait()
                zc.wait()
            else:
                bvec = bbuf[:]

                def last_body(i, _):
                    accb[i, :] = zbuf[i, :] + drepb[i, :] * accb[i, :] + bvec
                    return 0
                lax.fori_loop(0, NODES_T, last_body, 0)

                @pl.when(cid == 0)
                def _():
                    pltpu.sync_copy(accb, out_h.at[nsl])
            plsc.subcore_barrier()

    kfn = pl.kernel(body, out_type=out_type, mesh=mesh,
                    scratch_types=scratch,
                    compiler_params=pltpu.CompilerParams(
                        use_tc_tiling_on_sc=False))
    return kfn


BR = 2000


def _tc1_body(x_ref, w_ref, o0, o1, o2, o3):
    xs = x_ref[:]
    s = jnp.sum(xs, axis=1, keepdims=True)
    h = xs / jnp.clip(s, 1e-8, None)
    w = w_ref[:]
    outs = (o0, o1, o2, o3)
    for k in range(K + 1):
        outs[k][:] = jnp.dot(h, w[k], preferred_element_type=jnp.float32)


def _tc2_body(y_ref, b_ref, w_ref, o0, o1, o2, o3):
    h = jnp.maximum(y_ref[:] + b_ref[:], 0.0)
    w = w_ref[:]
    outs = (o0, o1, o2, o3)
    for k in range(K + 1):
        outs[k][:] = jnp.dot(h, w[k], preferred_element_type=jnp.float32)


def _tc1(x, W1):
    grid = N // BR
    return pl.pallas_call(
        _tc1_body,
        grid=(grid,),
        in_specs=[
            pl.BlockSpec((BR, DIN), lambda i: (i, 0)),
            pl.BlockSpec((K + 1, DIN, HID), lambda i: (0, 0, 0)),
        ],
        out_specs=[pl.BlockSpec((BR, HID), lambda i: (i, 0))] * (K + 1),
        out_shape=[jax.ShapeDtypeStruct((NPAD, HID), jnp.float32)] * (K + 1),
    )(x, W1)


def _tc2(y, b1, W2):
    grid = N // BR
    return pl.pallas_call(
        _tc2_body,
        grid=(grid,),
        in_specs=[
            pl.BlockSpec((BR, HID), lambda i: (i, 0)),
            pl.BlockSpec((1, HID), lambda i: (0, 0)),
            pl.BlockSpec((K + 1, HID, DOUT), lambda i: (0, 0, 0)),
        ],
        out_specs=[pl.BlockSpec((BR, DOUT), lambda i: (i, 0))] * (K + 1),
        out_shape=[jax.ShapeDtypeStruct((NPAD, DOUT), jnp.float32)] * (K + 1),
    )(y, b1, W2)


@jax.jit
def kernel(x, edge_index, W1, b1, W2, b2):
    row = edge_index[0].astype(jnp.int32).reshape(NT, NWIN, W_E)
    col = edge_index[1].astype(jnp.int32).reshape(NT, NWIN, W_E)

    z = _tc1(x, W1)
    sc1 = _make_sc_layer(compute_deg=True, fdim=HID)
    out1, _u1, drep = sc1(z[0], z[1], z[2], z[3], row, col,
                           jnp.zeros((16,), jnp.float32))

    z2 = _tc2(out1, b1.reshape(1, HID), W2)
    sc2 = _make_sc_layer(compute_deg=False, fdim=DOUT)
    out, _u2 = sc2(z2[0], z2[1], z2[2], z2[3], row, col, b2, drep)
    return out[:N]


# speedup vs baseline: 1.2544x; 1.2544x over previous
import functools

import jax
import jax.numpy as jnp
from jax import lax
from jax.experimental import pallas as pl
from jax.experimental.pallas import tpu as pltpu
from jax.experimental.pallas import tpu_sc as plsc

N = 10000
E = 320000
DIN = 128
HID = 16
DOUT = 16
K = 3

NT = 16
NPAD = 10240
NODES_T = NPAD // NT
EDGES_T = E // NT
W_E = 500
NWIN = EDGES_T // W_E
NBUF = 3


def _fill_rows(ref, n_rows, value):
    def body(i, _):
        ref[i, :] = jnp.full((16,), value, jnp.float32)
        return 0
    lax.fori_loop(0, n_rows, body, 0)


def _make_sc_layer(compute_deg: bool, fdim: int):
    mesh = plsc.VectorSubcoreMesh(core_axis_name="c", subcore_axis_name="s",
                                  num_cores=2, num_subcores=NT)
    out_type = [jax.ShapeDtypeStruct((NPAD, fdim), jnp.float32)]
    if compute_deg:
        out_type.append(jax.ShapeDtypeStruct((NPAD, 16), jnp.float32))

    scratch = [
        pltpu.VMEM((NWIN, W_E), jnp.int32),
        pltpu.VMEM((NWIN, W_E), jnp.int32),
        pltpu.VMEM((NBUF, W_E, 16), jnp.float32),
        pltpu.VMEM((NODES_T, 16), jnp.float32),
        pltpu.VMEM((NODES_T, 16), jnp.float32),
        pltpu.VMEM((NODES_T, 16), jnp.float32),
        pltpu.VMEM((NODES_T, 16), jnp.float32),
        pltpu.VMEM((16,), jnp.float32),
        pltpu.VMEM_SHARED((NPAD, 16), jnp.float32),
        pltpu.VMEM_SHARED((NPAD, 16), jnp.float32),
    ] + [pltpu.SemaphoreType.DMA] * (2 * NBUF + 1)

    def body(*refs):
        if compute_deg:
            (z0, z1, z2, z3, row_h, col_h, b_h,
             out_h, drep_h,
             row_v, col_v, gbufs, accb, zbuf, ubuf, drepb, bbuf,
             u_sp, acc_sp, *sems) = refs
        else:
            (z0, z1, z2, z3, row_h, col_h, b_h, drep_in,
             out_h,
             row_v, col_v, gbufs, accb, zbuf, ubuf, drepb, bbuf,
             u_sp, acc_sp, *sems) = refs
        zs = [z0, z1, z2, z3]
        gsems = sems[:NBUF]
        ssems = sems[NBUF:2 * NBUF]
        zsem = sems[2 * NBUF]
        cid = lax.axis_index("c")
        sid = lax.axis_index("s")
        base = sid * NODES_T
        nsl = pl.ds(base, NODES_T)

        rd = pltpu.async_copy(row_h.at[sid], row_v, gsems[0])
        cd = pltpu.async_copy(col_h.at[sid], col_v, gsems[1])
        bd = pltpu.async_copy(b_h, bbuf, zsem)
        if not compute_deg:
            dd = pltpu.async_copy(drep_in.at[nsl], drepb, ssems[0])

        _fill_rows(ubuf, NODES_T, 0.0)
        pltpu.sync_copy(ubuf, acc_sp.at[nsl])
        rd.wait()
        cd.wait()
        bd.wait()
        if not compute_deg:
            dd.wait()
        plsc.subcore_barrier()

        if compute_deg:
            ones = gbufs.at[0]
            _fill_rows(ones, W_E, 1.0)
            degd = [
                pltpu.async_copy(ones, acc_sp.at[col_v.at[j]], ssems[0],
                                 add=True)
                for j in range(NWIN)
            ]
            for d in degd:
                d.wait()
            plsc.subcore_barrier()
            pltpu.sync_copy(acc_sp.at[nsl], accb)

            def dinv_body(i, _):
                d = accb[i, :]
                bits = lax.bitcast_convert_type(d, jnp.int32)
                y = lax.bitcast_convert_type(
                    jnp.int32(0x5F3759DF) - (bits >> 1), jnp.float32)
                for _ in range(3):
                    y = y * (1.5 - 0.5 * d * y * y)
                drepb[i, :] = jnp.where(d > 0.5, y, 0.0)
                return 0
            lax.fori_loop(0, NODES_T, dinv_body, 0)

            @pl.when(cid == 0)
            def _():
                pltpu.sync_copy(drepb, drep_h.at[nsl])

            pltpu.sync_copy(ubuf, acc_sp.at[nsl])

        pltpu.sync_copy(zs[K].at[nsl], zbuf)

        def init_body(i, _):
            zbuf[i, :] = drepb[i, :] * zbuf[i, :]
            return 0
        lax.fori_loop(0, NODES_T, init_body, 0)
        pltpu.sync_copy(zbuf, u_sp.at[nsl])
        plsc.subcore_barrier()

        for k in range(K - 1, -1, -1):
            zd = pltpu.async_copy(zs[k].at[nsl], zbuf, zsem)
            gd = [None] * NWIN
            sd = [None] * NWIN
            for j in range(min(NBUF, NWIN)):
                gd[j] = pltpu.async_copy(
                    u_sp.at[row_v.at[j]], gbufs.at[j % NBUF], gsems[j % NBUF])
            for j in range(NWIN):
                b = j % NBUF
                gd[j].wait()
                sd[j] = pltpu.async_copy(
                    gbufs.at[b], acc_sp.at[col_v.at[j]], ssems[b], add=True)
                if j >= 1 and j + NBUF - 1 < NWIN:
                    sd[j - 1].wait()
                    nj = j + NBUF - 1
                    gd[nj] = pltpu.async_copy(
                        u_sp.at[row_v.at[nj]], gbufs.at[nj % NBUF],
                        gsems[nj % NBUF])
            for j in range(max(0, NWIN - NBUF), NWIN):
                sd[j].wait()
            plsc.subcore_barrier()

            pltpu.sync_copy(acc_sp.at[nsl], accb)
            zd.wait()
            if k > 0:
                def hop_body(i, _):
                    d = drepb[i, :]
                    accb[i, :] = d * (zbuf[i, :] + d * accb[i, :])
                    return 0
                lax.fori_loop(0, NODES_T, hop_body, 0)
                ud = pltpu.async_copy(accb, u_sp.at[nsl], gsems[0])
                zc = pltpu.async_copy(ubuf, acc_sp.at[nsl], gsems[1])
                ud.wait()
                zc.wait()
            else:
                bvec = bbuf[:]

                def last_body(i, _):
                    accb[i, :] = zbuf[i, :] + drepb[i, :] * accb[i, :] + bvec
                    return 0
                lax.fori_loop(0, NODES_T, last_body, 0)

                @pl.when(cid == 0)
                def _():
                    pltpu.sync_copy(accb, out_h.at[nsl])
            plsc.subcore_barrier()

    kfn = pl.kernel(body, out_type=out_type, mesh=mesh,
                    scratch_types=scratch,
                    compiler_params=pltpu.CompilerParams(
                        use_tc_tiling_on_sc=False))
    return kfn


BR = 2000


def _tc1_body(x_ref, w_ref, o0, o1, o2, o3):
    xs = x_ref[:]
    s = jnp.sum(xs, axis=1, keepdims=True)
    h = xs / jnp.clip(s, 1e-8, None)
    w = w_ref[:]
    outs = (o0, o1, o2, o3)
    for k in range(K + 1):
        outs[k][:] = jnp.dot(h, w[k], preferred_element_type=jnp.float32)


def _tc2_body(y_ref, b_ref, w_ref, o0, o1, o2, o3):
    h = jnp.maximum(y_ref[:] + b_ref[:], 0.0)
    w = w_ref[:]
    outs = (o0, o1, o2, o3)
    for k in range(K + 1):
        outs[k][:] = jnp.dot(h, w[k], preferred_element_type=jnp.float32)


def _tc1(x, W1):
    grid = N // BR
    return pl.pallas_call(
        _tc1_body,
        grid=(grid,),
        in_specs=[
            pl.BlockSpec((BR, DIN), lambda i: (i, 0)),
            pl.BlockSpec((K + 1, DIN, HID), lambda i: (0, 0, 0)),
        ],
        out_specs=[pl.BlockSpec((BR, HID), lambda i: (i, 0))] * (K + 1),
        out_shape=[jax.ShapeDtypeStruct((NPAD, HID), jnp.float32)] * (K + 1),
    )(x, W1)


def _tc2(y, b1, W2):
    grid = N // BR
    return pl.pallas_call(
        _tc2_body,
        grid=(grid,),
        in_specs=[
            pl.BlockSpec((BR, HID), lambda i: (i, 0)),
            pl.BlockSpec((1, HID), lambda i: (0, 0)),
            pl.BlockSpec((K + 1, HID, DOUT), lambda i: (0, 0, 0)),
        ],
        out_specs=[pl.BlockSpec((BR, DOUT), lambda i: (i, 0))] * (K + 1),
        out_shape=[jax.ShapeDtypeStruct((NPAD, DOUT), jnp.float32)] * (K + 1),
    )(y, b1, W2)


@jax.jit
def kernel(x, edge_index, W1, b1, W2, b2):
    row = edge_index[0].astype(jnp.int32).reshape(NT, NWIN, W_E)
    col = edge_index[1].astype(jnp.int32).reshape(NT, NWIN, W_E)

    z = _tc1(x, W1)
    sc1 = _make_sc_layer(compute_deg=True, fdim=HID)
    out1, drep = sc1(z[0], z[1], z[2], z[3], row, col,
                     jnp.zeros((16,), jnp.float32))

    z2 = _tc2(out1, b1.reshape(1, HID), W2)
    sc2 = _make_sc_layer(compute_deg=False, fdim=DOUT)
    (out,) = sc2(z2[0], z2[1], z2[2], z2[3], row, col, b2, drep)
    return out[:N]


# speedup vs baseline: 1.2750x; 1.0164x over previous
import functools

import jax
import jax.numpy as jnp
from jax import lax
from jax.experimental import pallas as pl
from jax.experimental.pallas import tpu as pltpu
from jax.experimental.pallas import tpu_sc as plsc

N = 10000
E = 320000
DIN = 128
HID = 16
DOUT = 16
K = 3

NT = 16
NPAD = 10240
NODES_T = NPAD // NT
EDGES_T = E // NT
W_E = 400
NWIN = EDGES_T // W_E
NBUF = 4


def _fill_rows(ref, n_rows, value):
    def body(i, _):
        ref[i, :] = jnp.full((16,), value, jnp.float32)
        return 0
    lax.fori_loop(0, n_rows, body, 0)


def _make_sc_layer(compute_deg: bool, fdim: int):
    mesh = plsc.VectorSubcoreMesh(core_axis_name="c", subcore_axis_name="s",
                                  num_cores=2, num_subcores=NT)
    out_type = [jax.ShapeDtypeStruct((NPAD, fdim), jnp.float32)]
    if compute_deg:
        out_type.append(jax.ShapeDtypeStruct((NPAD, 16), jnp.float32))

    scratch = [
        pltpu.VMEM((NWIN, W_E), jnp.int32),
        pltpu.VMEM((NWIN, W_E), jnp.int32),
        pltpu.VMEM((NBUF, W_E, 16), jnp.float32),
        pltpu.VMEM((NODES_T, 16), jnp.float32),
        pltpu.VMEM((NODES_T, 16), jnp.float32),
        pltpu.VMEM((NODES_T, 16), jnp.float32),
        pltpu.VMEM((NODES_T, 16), jnp.float32),
        pltpu.VMEM((16,), jnp.float32),
        pltpu.VMEM_SHARED((NPAD, 16), jnp.float32),
        pltpu.VMEM_SHARED((NPAD, 16), jnp.float32),
    ] + [pltpu.SemaphoreType.DMA] * (2 * NBUF + 1)

    def body(*refs):
        if compute_deg:
            (z0, z1, z2, z3, row_h, col_h, b_h,
             out_h, drep_h,
             row_v, col_v, gbufs, accb, zbuf, ubuf, drepb, bbuf,
             u_sp, acc_sp, *sems) = refs
        else:
            (z0, z1, z2, z3, row_h, col_h, b_h, drep_in,
             out_h,
             row_v, col_v, gbufs, accb, zbuf, ubuf, drepb, bbuf,
             u_sp, acc_sp, *sems) = refs
        zs = [z0, z1, z2, z3]
        gsems = sems[:NBUF]
        ssems = sems[NBUF:2 * NBUF]
        zsem = sems[2 * NBUF]
        cid = lax.axis_index("c")
        sid = lax.axis_index("s")
        base = sid * NODES_T
        nsl = pl.ds(base, NODES_T)

        rd = pltpu.async_copy(row_h.at[sid], row_v, gsems[0])
        cd = pltpu.async_copy(col_h.at[sid], col_v, gsems[1])
        bd = pltpu.async_copy(b_h, bbuf, zsem)
        if not compute_deg:
            dd = pltpu.async_copy(drep_in.at[nsl], drepb, ssems[0])

        _fill_rows(ubuf, NODES_T, 0.0)
        pltpu.sync_copy(ubuf, acc_sp.at[nsl])
        rd.wait()
        cd.wait()
        bd.wait()
        if not compute_deg:
            dd.wait()
        plsc.subcore_barrier()

        if compute_deg:
            ones = gbufs.at[0]
            _fill_rows(ones, W_E, 1.0)
            degd = [
                pltpu.async_copy(ones, acc_sp.at[col_v.at[j]], ssems[0],
                                 add=True)
                for j in range(NWIN)
            ]
            for d in degd:
                d.wait()
            plsc.subcore_barrier()
            pltpu.sync_copy(acc_sp.at[nsl], accb)

            def dinv_body(i, _):
                d = accb[i, :]
                bits = lax.bitcast_convert_type(d, jnp.int32)
                y = lax.bitcast_convert_type(
                    jnp.int32(0x5F3759DF) - (bits >> 1), jnp.float32)
                for _ in range(3):
                    y = y * (1.5 - 0.5 * d * y * y)
                drepb[i, :] = jnp.where(d > 0.5, y, 0.0)
                return 0
            lax.fori_loop(0, NODES_T, dinv_body, 0)

            @pl.when(cid == 0)
            def _():
                pltpu.sync_copy(drepb, drep_h.at[nsl])

            pltpu.sync_copy(ubuf, acc_sp.at[nsl])

        pltpu.sync_copy(zs[K].at[nsl], zbuf)

        def init_body(i, _):
            zbuf[i, :] = drepb[i, :] * zbuf[i, :]
            return 0
        lax.fori_loop(0, NODES_T, init_body, 0)
        pltpu.sync_copy(zbuf, u_sp.at[nsl])
        plsc.subcore_barrier()

        for k in range(K - 1, -1, -1):
            zd = pltpu.async_copy(zs[k].at[nsl], zbuf, zsem)
            gd = [None] * NWIN
            sd = [None] * NWIN
            for j in range(min(NBUF, NWIN)):
                gd[j] = pltpu.async_copy(
                    u_sp.at[row_v.at[j]], gbufs.at[j % NBUF], gsems[j % NBUF])
            for j in range(NWIN):
                b = j % NBUF
                gd[j].wait()
                sd[j] = pltpu.async_copy(
                    gbufs.at[b], acc_sp.at[col_v.at[j]], ssems[b], add=True)
                if j >= 1 and j + NBUF - 1 < NWIN:
                    sd[j - 1].wait()
                    nj = j + NBUF - 1
                    gd[nj] = pltpu.async_copy(
                        u_sp.at[row_v.at[nj]], gbufs.at[nj % NBUF],
                        gsems[nj % NBUF])
            for j in range(max(0, NWIN - NBUF), NWIN):
                sd[j].wait()
            plsc.subcore_barrier()

            pltpu.sync_copy(acc_sp.at[nsl], accb)
            zd.wait()
            if k > 0:
                def hop_body(i, _):
                    d = drepb[i, :]
                    accb[i, :] = d * (zbuf[i, :] + d * accb[i, :])
                    return 0
                lax.fori_loop(0, NODES_T, hop_body, 0)
                ud = pltpu.async_copy(accb, u_sp.at[nsl], gsems[0])
                zc = pltpu.async_copy(ubuf, acc_sp.at[nsl], gsems[1])
                ud.wait()
                zc.wait()
            else:
                bvec = bbuf[:]

                def last_body(i, _):
                    accb[i, :] = zbuf[i, :] + drepb[i, :] * accb[i, :] + bvec
                    return 0
                lax.fori_loop(0, NODES_T, last_body, 0)

                @pl.when(cid == 0)
                def _():
                    pltpu.sync_copy(accb, out_h.at[nsl])
            plsc.subcore_barrier()

    kfn = pl.kernel(body, out_type=out_type, mesh=mesh,
                    scratch_types=scratch,
                    compiler_params=pltpu.CompilerParams(
                        use_tc_tiling_on_sc=False))
    return kfn


BR = 2000


def _tc1_body(x_ref, w_ref, o0, o1, o2, o3):
    xs = x_ref[:]
    s = jnp.sum(xs, axis=1, keepdims=True)
    h = xs / jnp.clip(s, 1e-8, None)
    w = w_ref[:]
    outs = (o0, o1, o2, o3)
    for k in range(K + 1):
        outs[k][:] = jnp.dot(h, w[k], preferred_element_type=jnp.float32)


def _tc2_body(y_ref, b_ref, w_ref, o0, o1, o2, o3):
    h = jnp.maximum(y_ref[:] + b_ref[:], 0.0)
    w = w_ref[:]
    outs = (o0, o1, o2, o3)
    for k in range(K + 1):
        outs[k][:] = jnp.dot(h, w[k], preferred_element_type=jnp.float32)


def _tc1(x, W1):
    grid = N // BR
    return pl.pallas_call(
        _tc1_body,
        grid=(grid,),
        in_specs=[
            pl.BlockSpec((BR, DIN), lambda i: (i, 0)),
            pl.BlockSpec((K + 1, DIN, HID), lambda i: (0, 0, 0)),
        ],
        out_specs=[pl.BlockSpec((BR, HID), lambda i: (i, 0))] * (K + 1),
        out_shape=[jax.ShapeDtypeStruct((NPAD, HID), jnp.float32)] * (K + 1),
    )(x, W1)


def _tc2(y, b1, W2):
    grid = N // BR
    return pl.pallas_call(
        _tc2_body,
        grid=(grid,),
        in_specs=[
            pl.BlockSpec((BR, HID), lambda i: (i, 0)),
            pl.BlockSpec((1, HID), lambda i: (0, 0)),
            pl.BlockSpec((K + 1, HID, DOUT), lambda i: (0, 0, 0)),
        ],
        out_specs=[pl.BlockSpec((BR, DOUT), lambda i: (i, 0))] * (K + 1),
        out_shape=[jax.ShapeDtypeStruct((NPAD, DOUT), jnp.float32)] * (K + 1),
    )(y, b1, W2)


@jax.jit
def kernel(x, edge_index, W1, b1, W2, b2):
    row = edge_index[0].astype(jnp.int32).reshape(NT, NWIN, W_E)
    col = edge_index[1].astype(jnp.int32).reshape(NT, NWIN, W_E)

    z = _tc1(x, W1)
    sc1 = _make_sc_layer(compute_deg=True, fdim=HID)
    out1, drep = sc1(z[0], z[1], z[2], z[3], row, col,
                     jnp.zeros((16,), jnp.float32))

    z2 = _tc2(out1, b1.reshape(1, HID), W2)
    sc2 = _make_sc_layer(compute_deg=False, fdim=DOUT)
    (out,) = sc2(z2[0], z2[1], z2[2], z2[3], row, col, b2, drep)
    return out[:N]
